# baseline (speedup 1.0000x reference)
import jax
from jax.experimental import pallas as pl
from jax.experimental.pallas import tpu as pltpu


def _copy_kernel(x_ref, o_ref, sem):
    pltpu.make_async_copy(x_ref, o_ref, sem).start()
    pltpu.make_async_copy(x_ref, o_ref, sem).wait()


def kernel(x, edge_index):
    del edge_index
    return pl.pallas_call(
        _copy_kernel,
        out_shape=jax.ShapeDtypeStruct(x.shape, x.dtype),
        in_specs=[pl.BlockSpec(memory_space=pl.ANY)],
        out_specs=pl.BlockSpec(memory_space=pl.ANY),
        scratch_shapes=[pltpu.SemaphoreType.DMA],
    )(x)


# speedup vs baseline: 18.5407x; 18.5407x over previous
import jax
from jax.experimental import pallas as pl
from jax.experimental.pallas import tpu as pltpu

_BLOCK_ROWS = 1000


def _copy_kernel(x_ref, o_ref):
    o_ref[...] = x_ref[...]


def kernel(x, edge_index):
    del edge_index
    n, d = x.shape
    grid = (n // _BLOCK_ROWS,)
    return pl.pallas_call(
        _copy_kernel,
        out_shape=jax.ShapeDtypeStruct(x.shape, x.dtype),
        grid=grid,
        in_specs=[pl.BlockSpec((_BLOCK_ROWS, d), lambda i: (i, 0))],
        out_specs=pl.BlockSpec((_BLOCK_ROWS, d), lambda i: (i, 0)),
        compiler_params=pltpu.CompilerParams(
            dimension_semantics=("arbitrary",),
        ),
    )(x)
